# Initial kernel scaffold; baseline (speedup 1.0000x reference)
import jax
import jax.numpy as jnp
from jax.experimental import pallas as pl


def kernel(x, edge_index, Wq, bq, Wk, bk, Wv, bv, Wskip, bskip, Wbeta, ln_g, ln_b, Wout, bout):
    raise NotImplementedError("write your pallas kernel here")


# speedup vs baseline: 18.6797x; 18.6797x over previous
import functools

import jax
import jax.numpy as jnp
from jax import lax
from jax.experimental import pallas as pl
from jax.experimental.pallas import tpu as pltpu
from jax.experimental.pallas import tpu_sc as plsc

N = 10000
E = 320000
D = 128
H = 8
C = 16
L = 3
SCALE = 1.0 / (C ** 0.5)

NC = 2
NS = 16
NW = NC * NS
CH = 80
EPT = E // NW
NCH = EPT // CH
NDS = N // NC
TRASH = NDS
ACC_R = 5008
RPT = 312
RTAIL = ACC_R - RPT * NS
EPT_R = E // NS
NCH_R = EPT_R // CH
DACC_W = 40064

NB = 1000
EB = 2000


def _sc_mesh():
    return plsc.VectorSubcoreMesh(
        core_axis_name="c", subcore_axis_name="s", num_cores=NC, num_subcores=NS)


def _sc_gather_body(q_hbm, k_hbm, v_hbm, src_hbm, dst_hbm, qd_hbm, ks_hbm, vs_hbm,
                    idx_s, idx_d, qbuf, kbuf, vbuf, sem):
    cid = lax.axis_index("c")
    sid = lax.axis_index("s")
    wid = sid * NC + cid
    base0 = wid * EPT

    def step(j, carry):
        base = base0 + j * CH
        pltpu.sync_copy(dst_hbm.at[pl.ds(base, CH)], idx_d)
        pltpu.sync_copy(src_hbm.at[pl.ds(base, CH)], idx_s)
        cq = pltpu.async_copy(q_hbm.at[idx_d], qbuf, sem)
        ck = pltpu.async_copy(k_hbm.at[idx_s], kbuf, sem)
        cv = pltpu.async_copy(v_hbm.at[idx_s], vbuf, sem)
        cq.wait()
        ck.wait()
        cv.wait()
        pltpu.sync_copy(qbuf, qd_hbm.at[pl.ds(base, CH)])
        pltpu.sync_copy(kbuf, ks_hbm.at[pl.ds(base, CH)])
        pltpu.sync_copy(vbuf, vs_hbm.at[pl.ds(base, CH)])
        return carry

    lax.fori_loop(0, NCH, step, 0)


@functools.cache
def _sc_gather():
    return pl.kernel(
        _sc_gather_body,
        out_type=(
            jax.ShapeDtypeStruct((E, D), jnp.float32),
            jax.ShapeDtypeStruct((E, D), jnp.float32),
            jax.ShapeDtypeStruct((E, D), jnp.float32),
        ),
        mesh=_sc_mesh(),
        scratch_types=[
            pltpu.VMEM((CH,), jnp.int32),
            pltpu.VMEM((CH,), jnp.int32),
            pltpu.VMEM((CH, D), jnp.float32),
            pltpu.VMEM((CH, D), jnp.float32),
            pltpu.VMEM((CH, D), jnp.float32),
            pltpu.SemaphoreType.DMA,
        ],
    )


def _sc_scatter_body(mrow_hbm, exf_hbm, dst_hbm, zrows_hbm,
                     out_hbm, den_hbm, idx_b, idx2_b, mbuf, exbuf, dacc, acc):
    cid = lax.axis_index("c")
    sid = lax.axis_index("s")
    pltpu.sync_copy(zrows_hbm, acc.at[pl.ds(sid * RPT, RPT)])

    @pl.when(sid == NS - 1)
    def _():
        pltpu.sync_copy(zrows_hbm.at[pl.ds(0, RTAIL)],
                        acc.at[pl.ds(NS * RPT, RTAIL)])

    z16 = jnp.zeros((16,), jnp.float32)

    def zstep(t, carry):
        for u in range(8):
            dacc[pl.ds((8 * t + u) * 16, 16)] = z16
        return carry

    lax.fori_loop(0, DACC_W // 128, zstep, 0)
    plsc.subcore_barrier()

    lane = lax.iota(jnp.int32, 16)
    cols_base = lane & 7
    m_lo = lane < 8
    m_hi = lane >= 8
    nbase = cid * NDS
    idx_row = idx_b.at[0]
    idx2_row = idx2_b.at[0]

    def step(j, carry):
        base = sid * EPT_R + j * CH
        base8 = sid * (EPT_R * H) + j * (CH * H)
        pltpu.sync_copy(dst_hbm.at[pl.ds(base, CH)], idx_row)
        pltpu.sync_copy(mrow_hbm.at[pl.ds(base, CH)], mbuf)
        pltpu.sync_copy(exf_hbm.at[pl.ds(base8, CH * H)], exbuf)
        for t in range(CH // 16):
            d = idx_row[pl.ds(t * 16, 16)]
            r = d - nbase
            ok = (r >= 0) & (r < NDS)
            idx2_row[pl.ds(t * 16, 16)] = jnp.where(ok, r, TRASH)
        pltpu.sync_copy(mbuf, acc.at[idx2_row], add=True)

        def pair(t, c):
            ex_v = exbuf[pl.ds(t * 16, 16)]
            r0 = plsc.load_gather(idx2_row, [jnp.full((16,), 2 * t, jnp.int32)])
            plsc.addupdate_scatter(dacc, [r0 * 8 + cols_base], ex_v, mask=m_lo)
            r1 = plsc.load_gather(idx2_row, [jnp.full((16,), 2 * t + 1, jnp.int32)])
            plsc.addupdate_scatter(dacc, [r1 * 8 + cols_base], ex_v, mask=m_hi)
            return c

        lax.fori_loop(0, CH // 2, pair, 0)
        return carry

    lax.fori_loop(0, NCH_R, step, 0)
    plsc.subcore_barrier()
    pltpu.sync_copy(acc.at[pl.ds(sid * RPT, RPT)],
                    out_hbm.at[cid, pl.ds(sid * RPT, RPT)])

    @pl.when(sid == NS - 1)
    def _():
        pltpu.sync_copy(acc.at[pl.ds(NS * RPT, RTAIL)],
                        out_hbm.at[cid, pl.ds(NS * RPT, RTAIL)])

    pltpu.sync_copy(dacc, den_hbm.at[cid, sid])


@functools.cache
def _sc_scatter():
    return pl.kernel(
        _sc_scatter_body,
        out_type=(
            jax.ShapeDtypeStruct((NC, ACC_R, D), jnp.float32),
            jax.ShapeDtypeStruct((NC, NS, DACC_W), jnp.float32),
        ),
        mesh=_sc_mesh(),
        compiler_params=pltpu.CompilerParams(needs_layout_passes=False),
        scratch_types=[
            pltpu.VMEM((1, CH), jnp.int32),
            pltpu.VMEM((1, CH), jnp.int32),
            pltpu.VMEM((CH, D), jnp.float32),
            pltpu.VMEM((CH * H,), jnp.float32),
            pltpu.VMEM((DACC_W,), jnp.float32),
            pltpu.VMEM_SHARED((ACC_R, D), jnp.float32),
        ],
    )


def _proj_body(h_ref, wq_ref, wk_ref, wv_ref, ws_ref,
               bq_ref, bk_ref, bv_ref, bs_ref,
               q_ref, k_ref, v_ref, xr_ref):
    h = h_ref[...]
    q_ref[...] = jnp.dot(h, wq_ref[...], preferred_element_type=jnp.float32) + bq_ref[...]
    k_ref[...] = jnp.dot(h, wk_ref[...], preferred_element_type=jnp.float32) + bk_ref[...]
    v_ref[...] = jnp.dot(h, wv_ref[...], preferred_element_type=jnp.float32) + bv_ref[...]
    xr_ref[...] = jnp.dot(h, ws_ref[...], preferred_element_type=jnp.float32) + bs_ref[...]


def _proj(h, wqt, wkt, wvt, wst, bq2, bk2, bv2, bs2):
    w_spec = pl.BlockSpec((D, D), lambda i: (0, 0))
    b_spec = pl.BlockSpec((1, D), lambda i: (0, 0))
    n_spec = pl.BlockSpec((NB, D), lambda i: (i, 0))
    return pl.pallas_call(
        _proj_body,
        grid=(N // NB,),
        in_specs=[n_spec, w_spec, w_spec, w_spec, w_spec,
                  b_spec, b_spec, b_spec, b_spec],
        out_specs=[n_spec, n_spec, n_spec, n_spec],
        out_shape=[jax.ShapeDtypeStruct((N, D), jnp.float32)] * 4,
    )(h, wqt, wkt, wvt, wst, bq2, bk2, bv2, bs2)


def _msg_body(qd_ref, ks_ref, vs_ref, m_ref, ex_ref):
    qd = qd_ref[...]
    ks = ks_ref[...]
    vs = vs_ref[...]
    parts = []
    exs = []
    for h in range(H):
        sl = slice(h * C, (h + 1) * C)
        t = jnp.sum(qd[:, sl] * ks[:, sl], axis=1, keepdims=True) * SCALE
        exh = jnp.exp(t)
        parts.append(vs[:, sl] * exh)
        exs.append(exh)
    m_ref[...] = jnp.concatenate(parts, axis=1)
    ex_ref[...] = jnp.concatenate(exs, axis=1)


def _msg(qd, ks, vs):
    e_spec = pl.BlockSpec((EB, D), lambda i: (i, 0))
    return pl.pallas_call(
        _msg_body,
        grid=(E // EB,),
        in_specs=[e_spec, e_spec, e_spec],
        out_specs=[e_spec, pl.BlockSpec((EB, H), lambda i: (i, 0))],
        out_shape=[jax.ShapeDtypeStruct((E, D), jnp.float32),
                   jax.ShapeDtypeStruct((E, H), jnp.float32)],
    )(qd, ks, vs)


def _combine_body(acc_ref, den_ref, xr_ref, res_ref, wb_ref, g_ref, b_ref, o_ref):
    acc = acc_ref[0]
    den = jnp.sum(den_ref[0], axis=0)
    outs = []
    for h in range(H):
        outs.append(acc[:, h * C:(h + 1) * C] / (den[:, h:h + 1] + 1e-16))
    out = jnp.concatenate(outs, axis=1)
    xr = xr_ref[...]
    wb = wb_ref[...]
    s = jnp.sum(out * wb[0:1] + xr * wb[1:2] + (out - xr) * wb[2:3],
                axis=1, keepdims=True)
    beta = 1.0 / (1.0 + jnp.exp(-s))
    y = beta * xr + (1.0 - beta) * out
    y = jnp.maximum(y, 0.0) + res_ref[...]
    m = jnp.mean(y, axis=1, keepdims=True)
    var = jnp.mean((y - m) ** 2, axis=1, keepdims=True)
    o_ref[...] = (y - m) * lax.rsqrt(var + 1e-5) * g_ref[...] + b_ref[...]


def _combine(accm, dens, xr, res, wb, g2, b2):
    nhalf = NDS // NB
    n_spec = pl.BlockSpec((NB, D), lambda i: (i, 0))
    return pl.pallas_call(
        _combine_body,
        grid=(N // NB,),
        in_specs=[pl.BlockSpec((1, NB, D), lambda i: (i // nhalf, i % nhalf, 0)),
                  pl.BlockSpec((1, NS, NB, H),
                               lambda i: (i // nhalf, 0, i % nhalf, 0)),
                  n_spec, n_spec,
                  pl.BlockSpec((3, D), lambda i: (0, 0)),
                  pl.BlockSpec((1, D), lambda i: (0, 0)),
                  pl.BlockSpec((1, D), lambda i: (0, 0))],
        out_specs=n_spec,
        out_shape=jax.ShapeDtypeStruct((N, D), jnp.float32),
    )(accm, dens, xr, res, wb, g2, b2)


def _final_body(h_ref, w_ref, b_ref, o_ref):
    o_ref[...] = jnp.dot(h_ref[...], w_ref[...],
                         preferred_element_type=jnp.float32) + b_ref[...]


def _final(h, wt, b2):
    n_spec = pl.BlockSpec((NB, D), lambda i: (i, 0))
    return pl.pallas_call(
        _final_body,
        grid=(N // NB,),
        in_specs=[n_spec,
                  pl.BlockSpec((D, D), lambda i: (0, 0)),
                  pl.BlockSpec((1, D), lambda i: (0, 0))],
        out_specs=n_spec,
        out_shape=jax.ShapeDtypeStruct((N, D), jnp.float32),
    )(h, wt, b2)


def kernel(x, edge_index, Wq, bq, Wk, bk, Wv, bv, Wskip, bskip, Wbeta,
           ln_g, ln_b, Wout, bout):
    src = edge_index[0]
    dst = edge_index[1]
    zrows = jnp.zeros((RPT, D), jnp.float32)
    h = x
    for l in range(L):
        q, k, v, xr = _proj(h, Wq[l].T, Wk[l].T, Wv[l].T, Wskip[l].T,
                            bq[l][None], bk[l][None], bv[l][None], bskip[l][None])
        qd, ks, vs = _sc_gather()(q, k, v, src, dst)
        mrow, exmat = _msg(qd, ks, vs)
        accm, dens = _sc_scatter()(mrow, exmat.reshape(E * H), dst, zrows)
        accm_n = accm[:, :NDS]
        dens_n = dens[:, :, :NDS * H].reshape(NC, NS, NDS, H)
        h = _combine(accm_n, dens_n, xr, h,
                     Wbeta[l][0].reshape(3, D), ln_g[l][None], ln_b[l][None])
    return _final(h, Wout.T, bout[None])


# speedup vs baseline: 22.0991x; 1.1831x over previous
import functools

import jax
import jax.numpy as jnp
from jax import lax
from jax.experimental import pallas as pl
from jax.experimental.pallas import tpu as pltpu
from jax.experimental.pallas import tpu_sc as plsc

N = 10000
E = 320000
D = 128
H = 8
C = 16
L = 3
SCALE = 1.0 / (C ** 0.5)

NC = 2
NS = 16
NW = NC * NS
CH = 80
EPT = E // NW
NCH = EPT // CH
ACC_R = 10112
RPT = ACC_R // NS
DEN_W = N * H

NB = 1000
EB = 2000


def _sc_mesh():
    return plsc.VectorSubcoreMesh(
        core_axis_name="c", subcore_axis_name="s", num_cores=NC, num_subcores=NS)


def _sc_gather_body(q_hbm, k_hbm, v_hbm, src_hbm, dst_hbm, qd_hbm, ks_hbm, vs_hbm,
                    idx_s, idx_d, qbuf, kbuf, vbuf, sem):
    cid = lax.axis_index("c")
    sid = lax.axis_index("s")
    wid = sid * NC + cid
    base0 = wid * EPT

    def step(j, carry):
        base = base0 + j * CH
        pltpu.sync_copy(dst_hbm.at[pl.ds(base, CH)], idx_d)
        pltpu.sync_copy(src_hbm.at[pl.ds(base, CH)], idx_s)
        cq = pltpu.async_copy(q_hbm.at[idx_d], qbuf, sem)
        ck = pltpu.async_copy(k_hbm.at[idx_s], kbuf, sem)
        cv = pltpu.async_copy(v_hbm.at[idx_s], vbuf, sem)
        cq.wait()
        ck.wait()
        cv.wait()
        pltpu.sync_copy(qbuf, qd_hbm.at[pl.ds(base, CH)])
        pltpu.sync_copy(kbuf, ks_hbm.at[pl.ds(base, CH)])
        pltpu.sync_copy(vbuf, vs_hbm.at[pl.ds(base, CH)])
        return carry

    lax.fori_loop(0, NCH, step, 0)


@functools.cache
def _sc_gather():
    return pl.kernel(
        _sc_gather_body,
        out_type=(
            jax.ShapeDtypeStruct((E, D), jnp.float32),
            jax.ShapeDtypeStruct((E, D), jnp.float32),
            jax.ShapeDtypeStruct((E, D), jnp.float32),
        ),
        mesh=_sc_mesh(),
        scratch_types=[
            pltpu.VMEM((CH,), jnp.int32),
            pltpu.VMEM((CH,), jnp.int32),
            pltpu.VMEM((CH, D), jnp.float32),
            pltpu.VMEM((CH, D), jnp.float32),
            pltpu.VMEM((CH, D), jnp.float32),
            pltpu.SemaphoreType.DMA,
        ],
    )


def _sc_scatmsg_body(mrow_hbm, dst_hbm, zrows_hbm, out_hbm, idx_b, mbuf, acc):
    cid = lax.axis_index("c")
    sid = lax.axis_index("s")
    pltpu.sync_copy(zrows_hbm, acc.at[pl.ds(sid * RPT, RPT)])
    plsc.subcore_barrier()

    idx_row = idx_b.at[0]
    base0 = (cid * NS + sid) * EPT

    def step(j, carry):
        base = base0 + j * CH
        pltpu.sync_copy(dst_hbm.at[pl.ds(base, CH)], idx_row)
        pltpu.sync_copy(mrow_hbm.at[pl.ds(base, CH)], mbuf)
        pltpu.sync_copy(mbuf, acc.at[idx_row], add=True)
        return carry

    lax.fori_loop(0, NCH, step, 0)
    plsc.subcore_barrier()
    pltpu.sync_copy(acc.at[pl.ds(sid * RPT, RPT)],
                    out_hbm.at[cid, pl.ds(sid * RPT, RPT)])


@functools.cache
def _sc_scatmsg():
    return pl.kernel(
        _sc_scatmsg_body,
        out_type=jax.ShapeDtypeStruct((NC, ACC_R, D), jnp.float32),
        mesh=_sc_mesh(),
        compiler_params=pltpu.CompilerParams(needs_layout_passes=False),
        scratch_types=[
            pltpu.VMEM((1, CH), jnp.int32),
            pltpu.VMEM((CH, D), jnp.float32),
            pltpu.VMEM_SHARED((ACC_R, D), jnp.float32),
        ],
    )


def _sc_scatden_body(exf_hbm, dst_hbm, zden_hbm, den_hbm, idx_b, exbuf, dacc):
    cid = lax.axis_index("c")
    sid = lax.axis_index("s")
    pltpu.sync_copy(zden_hbm, dacc)

    lane = lax.iota(jnp.int32, 16)
    cols_base = lane & 7
    m_lo = lane < 8
    m_hi = lane >= 8
    idx_row = idx_b.at[0]
    base0 = (cid * NS + sid) * EPT

    def step(j, carry):
        base = base0 + j * CH
        pltpu.sync_copy(dst_hbm.at[pl.ds(base, CH)], idx_row)
        pltpu.sync_copy(exf_hbm.at[pl.ds(base * H, CH * H)], exbuf)

        def pair(t, c):
            ex_v = exbuf[pl.ds(t * 16, 16)]
            r0 = plsc.load_gather(idx_row, [jnp.full((16,), 2 * t, jnp.int32)])
            plsc.addupdate_scatter(dacc, [r0 * 8 + cols_base], ex_v, mask=m_lo)
            r1 = plsc.load_gather(idx_row, [jnp.full((16,), 2 * t + 1, jnp.int32)])
            plsc.addupdate_scatter(dacc, [r1 * 8 + cols_base], ex_v, mask=m_hi)
            return c

        lax.fori_loop(0, CH // 2, pair, 0)
        return carry

    lax.fori_loop(0, NCH, step, 0)
    pltpu.sync_copy(dacc, den_hbm.at[cid, sid])


@functools.cache
def _sc_scatden():
    return pl.kernel(
        _sc_scatden_body,
        out_type=jax.ShapeDtypeStruct((NC, NS, DEN_W), jnp.float32),
        mesh=_sc_mesh(),
        compiler_params=pltpu.CompilerParams(needs_layout_passes=False),
        scratch_types=[
            pltpu.VMEM((1, CH), jnp.int32),
            pltpu.VMEM((CH * H,), jnp.float32),
            pltpu.VMEM((DEN_W,), jnp.float32),
        ],
    )


def _proj_body(h_ref, wq_ref, wk_ref, wv_ref, ws_ref,
               bq_ref, bk_ref, bv_ref, bs_ref,
               q_ref, k_ref, v_ref, xr_ref):
    h = h_ref[...]
    q_ref[...] = jnp.dot(h, wq_ref[...], preferred_element_type=jnp.float32) + bq_ref[...]
    k_ref[...] = jnp.dot(h, wk_ref[...], preferred_element_type=jnp.float32) + bk_ref[...]
    v_ref[...] = jnp.dot(h, wv_ref[...], preferred_element_type=jnp.float32) + bv_ref[...]
    xr_ref[...] = jnp.dot(h, ws_ref[...], preferred_element_type=jnp.float32) + bs_ref[...]


def _proj(h, wqt, wkt, wvt, wst, bq2, bk2, bv2, bs2):
    w_spec = pl.BlockSpec((D, D), lambda i: (0, 0))
    b_spec = pl.BlockSpec((1, D), lambda i: (0, 0))
    n_spec = pl.BlockSpec((NB, D), lambda i: (i, 0))
    return pl.pallas_call(
        _proj_body,
        grid=(N // NB,),
        in_specs=[n_spec, w_spec, w_spec, w_spec, w_spec,
                  b_spec, b_spec, b_spec, b_spec],
        out_specs=[n_spec, n_spec, n_spec, n_spec],
        out_shape=[jax.ShapeDtypeStruct((N, D), jnp.float32)] * 4,
    )(h, wqt, wkt, wvt, wst, bq2, bk2, bv2, bs2)


def _msg_body(qd_ref, ks_ref, vs_ref, m_ref, ex_ref):
    qd = qd_ref[...]
    ks = ks_ref[...]
    vs = vs_ref[...]
    parts = []
    exs = []
    for h in range(H):
        sl = slice(h * C, (h + 1) * C)
        t = jnp.sum(qd[:, sl] * ks[:, sl], axis=1, keepdims=True) * SCALE
        exh = jnp.exp(t)
        parts.append(vs[:, sl] * exh)
        exs.append(exh)
    m_ref[...] = jnp.concatenate(parts, axis=1)
    ex_ref[...] = jnp.concatenate(exs, axis=1)


def _msg(qd, ks, vs):
    e_spec = pl.BlockSpec((EB, D), lambda i: (i, 0))
    return pl.pallas_call(
        _msg_body,
        grid=(E // EB,),
        in_specs=[e_spec, e_spec, e_spec],
        out_specs=[e_spec, pl.BlockSpec((EB, H), lambda i: (i, 0))],
        out_shape=[jax.ShapeDtypeStruct((E, D), jnp.float32),
                   jax.ShapeDtypeStruct((E, H), jnp.float32)],
    )(qd, ks, vs)


def _combine_body(acc_ref, den_ref, xr_ref, res_ref, wb_ref, g_ref, b_ref, o_ref):
    acc = acc_ref[0] + acc_ref[1]
    den = jnp.sum(den_ref[...], axis=0)
    outs = []
    for h in range(H):
        outs.append(acc[:, h * C:(h + 1) * C] / (den[:, h:h + 1] + 1e-16))
    out = jnp.concatenate(outs, axis=1)
    xr = xr_ref[...]
    wb = wb_ref[...]
    s = jnp.sum(out * wb[0:1] + xr * wb[1:2] + (out - xr) * wb[2:3],
                axis=1, keepdims=True)
    beta = 1.0 / (1.0 + jnp.exp(-s))
    y = beta * xr + (1.0 - beta) * out
    y = jnp.maximum(y, 0.0) + res_ref[...]
    m = jnp.mean(y, axis=1, keepdims=True)
    var = jnp.mean((y - m) ** 2, axis=1, keepdims=True)
    o_ref[...] = (y - m) * lax.rsqrt(var + 1e-5) * g_ref[...] + b_ref[...]


def _combine(accm, den, xr, res, wb, g2, b2):
    n_spec = pl.BlockSpec((NB, D), lambda i: (i, 0))
    return pl.pallas_call(
        _combine_body,
        grid=(N // NB,),
        in_specs=[pl.BlockSpec((NC, NB, D), lambda i: (0, i, 0)),
                  pl.BlockSpec((NW, NB, H), lambda i: (0, i, 0)),
                  n_spec, n_spec,
                  pl.BlockSpec((3, D), lambda i: (0, 0)),
                  pl.BlockSpec((1, D), lambda i: (0, 0)),
                  pl.BlockSpec((1, D), lambda i: (0, 0))],
        out_specs=n_spec,
        out_shape=jax.ShapeDtypeStruct((N, D), jnp.float32),
    )(accm, den, xr, res, wb, g2, b2)


def _final_body(h_ref, w_ref, b_ref, o_ref):
    o_ref[...] = jnp.dot(h_ref[...], w_ref[...],
                         preferred_element_type=jnp.float32) + b_ref[...]


def _final(h, wt, b2):
    n_spec = pl.BlockSpec((NB, D), lambda i: (i, 0))
    return pl.pallas_call(
        _final_body,
        grid=(N // NB,),
        in_specs=[n_spec,
                  pl.BlockSpec((D, D), lambda i: (0, 0)),
                  pl.BlockSpec((1, D), lambda i: (0, 0))],
        out_specs=n_spec,
        out_shape=jax.ShapeDtypeStruct((N, D), jnp.float32),
    )(h, wt, b2)


def kernel(x, edge_index, Wq, bq, Wk, bk, Wv, bv, Wskip, bskip, Wbeta,
           ln_g, ln_b, Wout, bout):
    src = edge_index[0]
    dst = edge_index[1]
    zrows = jnp.zeros((RPT, D), jnp.float32)
    zden = jnp.zeros((DEN_W,), jnp.float32)
    h = x
    for l in range(L):
        q, k, v, xr = _proj(h, Wq[l].T, Wk[l].T, Wv[l].T, Wskip[l].T,
                            bq[l][None], bk[l][None], bv[l][None], bskip[l][None])
        qd, ks, vs = _sc_gather()(q, k, v, src, dst)
        mrow, ex = _msg(qd, ks, vs)
        accm = _sc_scatmsg()(mrow, dst, zrows)
        den = _sc_scatden()(ex.reshape(E * H), dst, zden)
        h = _combine(accm[:, :N], den.reshape(NW, N, H), xr, h,
                     Wbeta[l][0].reshape(3, D), ln_g[l][None], ln_b[l][None])
    return _final(h, Wout.T, bout[None])
